# baseline (speedup 1.0000x reference)
import functools

import jax
import jax.numpy as jnp
from jax import lax
from jax.experimental import pallas as pl
from jax.experimental.pallas import tpu as pltpu
from jax.experimental.pallas import tpu_sc as plsc

_B = 256; _T = 8; _D = 512; _NL = 2; _E = 8; _F = 1024
_VOCAB = 119; _FEAT = 200; _PBF = 16; _PB_SCALE = 10.0
_N = _B * _T
_G = 256
_NBLK = 24
_NPAD = _NBLK * _G
_NW = 32


def _ln_rows(x, g, b):
    mu = jnp.mean(x, axis=-1, keepdims=True)
    xc = x - mu
    var = jnp.mean(xc * xc, axis=-1, keepdims=True)
    return xc * jax.lax.rsqrt(var + 1e-5) * g + b


def _core(x, pbv, kw_ref, kb_ref, vw_ref, vb_ref, ow_ref, ob_ref,
          g1_ref, b1_ref, gw_ref, gb_ref,
          x1_ref, pos0_ref, pos1_ref, s1_ref, s2_ref, be_ref, nv_ref):
    xb = x.astype(jnp.bfloat16)
    K = jnp.dot(xb, kw_ref[...].astype(jnp.bfloat16),
                preferred_element_type=jnp.float32) + kb_ref[...]
    V = jnp.dot(xb, vw_ref[...].astype(jnp.bfloat16),
                preferred_element_type=jnp.float32) + vb_ref[...]
    K3 = K.reshape(_B, _T, _D)
    V3 = V.reshape(_B, _T, _D)
    pb3 = pbv.reshape(_B, _T, _T)
    wv3 = jnp.broadcast_to(jnp.sum(K3 * V3, axis=1, keepdims=True), (_B, _T, _D))
    for jj in range(_T):
        wv3 = wv3 + pb3[:, :, jj:jj + 1] * V3[:, jj:jj + 1, :]
    wv = wv3.reshape(_N, _D).astype(jnp.bfloat16)
    attn = jnp.dot(wv, ow_ref[...].astype(jnp.bfloat16),
                   preferred_element_type=jnp.float32) + ob_ref[...]
    xn = _ln_rows(x + attn, g1_ref[...], b1_ref[...])
    x1_ref[...] = xn

    logits = jnp.dot(xn, gw_ref[...], preferred_element_type=jnp.float32) + gb_ref[...]
    m = jnp.max(logits, axis=-1, keepdims=True)
    ex = jnp.exp(logits - m)
    p = ex / jnp.sum(ex, axis=-1, keepdims=True)
    lane = jax.lax.broadcasted_iota(jnp.int32, (_N, _E), 1)
    m1 = jnp.max(p, axis=-1, keepdims=True)
    i1 = jnp.min(jnp.where(p == m1, lane, _E), axis=-1, keepdims=True)
    oh1 = lane == i1
    pm = jnp.where(oh1, -1.0, p)
    m2 = jnp.max(pm, axis=-1, keepdims=True)
    i2 = jnp.min(jnp.where(pm == m2, lane, _E), axis=-1, keepdims=True)
    oh2 = lane == i2
    s1_ref[...] = m1
    s2_ref[...] = m2

    m0f = jnp.where(oh1, 1.0, 0.0)
    m1f = jnp.where(oh2, 1.0, 0.0)
    mm = jnp.concatenate([m0f, m1f], axis=1).astype(jnp.bfloat16)
    tri = (jax.lax.broadcasted_iota(jnp.int32, (_N, _N), 0)
           >= jax.lax.broadcasted_iota(jnp.int32, (_N, _N), 1)).astype(jnp.bfloat16)
    cum = jnp.dot(tri, mm, preferred_element_type=jnp.float32)
    cum0 = cum[:, :_E]
    cum1 = cum[:, _E:]
    c0 = cum0[_N - 1:_N, :]
    ct = c0 + cum1[_N - 1:_N, :]
    cti = ct.astype(jnp.int32)
    cpad = (jax.lax.shift_right_logical(cti + (_G - 1), 8) * _G).astype(jnp.float32)
    e0 = jax.lax.broadcasted_iota(jnp.int32, (_E, _E), 0)
    e1 = jax.lax.broadcasted_iota(jnp.int32, (_E, _E), 1)
    tri8x = (e0 < e1).astype(jnp.float32)
    tri8i = (e0 <= e1).astype(jnp.float32)
    bs = jnp.dot(cpad, tri8x, preferred_element_type=jnp.float32)
    pos0 = jnp.sum(m0f * (cum0 - 1.0 + bs), axis=-1, keepdims=True)
    pos1 = jnp.sum(m1f * (cum1 - 1.0 + bs + c0), axis=-1, keepdims=True)
    pos0_ref[...] = pos0.astype(jnp.int32)
    pos1_ref[...] = pos1.astype(jnp.int32)

    bcnt = cpad * (1.0 / _G)
    ib = jnp.dot(bcnt, tri8i, preferred_element_type=jnp.float32)
    ba = jax.lax.broadcasted_iota(jnp.int32, (_NBLK, 1), 0).astype(jnp.float32)
    ge = (ba >= jnp.broadcast_to(ib, (_NBLK, _E))).astype(jnp.float32)
    be = jnp.sum(ge, axis=-1, keepdims=True)
    be_i = jnp.minimum(be, float(_E - 1)).astype(jnp.int32)
    lane8 = jax.lax.broadcasted_iota(jnp.int32, (_NBLK, _E), 1)
    ohbe = lane8 == be_i
    startb = ib - bcnt
    sbe = jnp.sum(jnp.where(ohbe, jnp.broadcast_to(startb, (_NBLK, _E)), 0.0),
                  axis=-1, keepdims=True)
    cbe = jnp.sum(jnp.where(ohbe, jnp.broadcast_to(ct, (_NBLK, _E)), 0.0),
                  axis=-1, keepdims=True)
    nv = jnp.clip(cbe - (ba - sbe) * _G, 0.0, float(_G))
    nv = jnp.where(be >= float(_E), 0.0, nv)
    be_ref[...] = be_i
    nv_ref[...] = nv.astype(jnp.int32)


def _combine_ln2(xprev_ref, eo0_ref, eo1_ref, s1p_ref, s2p_ref, g2_ref, b2_ref):
    moe = s1p_ref[...] * eo0_ref[...] + s2p_ref[...] * eo1_ref[...]
    return _ln_rows(xprev_ref[...] + moe, g2_ref[...], b2_ref[...])


def _a0_first_body(pbW_s, pbb_s, pba_s,
                   src_ref, friT_ref, frjT_ref, cbfv_ref, Wm_ref, bm_ref,
                   kw_ref, kb_ref, vw_ref, vb_ref, ow_ref, ob_ref,
                   g1_ref, b1_ref, gw_ref, gb_ref,
                   x1_ref, pos0_ref, pos1_ref, s1_ref, s2_ref,
                   be_ref, nv_ref, pb_ref):
    cw = jnp.dot(cbfv_ref[...], Wm_ref[...], preferred_element_type=jnp.float32)
    oh = (src_ref[...] == jax.lax.broadcasted_iota(jnp.int32, (_N, _VOCAB), 1)
          ).astype(jnp.float32)
    x = jnp.dot(oh, cw, preferred_element_type=jnp.float32) + bm_ref[...]
    dT = (frjT_ref[...] - friT_ref[...]) * _PB_SCALE
    acc = jnp.zeros((_T, _N), jnp.float32)
    for f in range(_PBF):
        acc = acc + jnp.cos(dT * pbW_s[f] + pbb_s[f]) * pba_s[f]
    pbv = acc.T
    pb_ref[...] = pbv
    _core(x, pbv, kw_ref, kb_ref, vw_ref, vb_ref, ow_ref, ob_ref,
          g1_ref, b1_ref, gw_ref, gb_ref,
          x1_ref, pos0_ref, pos1_ref, s1_ref, s2_ref, be_ref, nv_ref)


def _a0_next_body(xprev_ref, eo0_ref, eo1_ref, s1p_ref, s2p_ref,
                  g2p_ref, b2p_ref, pbin_ref,
                  kw_ref, kb_ref, vw_ref, vb_ref, ow_ref, ob_ref,
                  g1_ref, b1_ref, gw_ref, gb_ref,
                  x1_ref, pos0_ref, pos1_ref, s1_ref, s2_ref,
                  be_ref, nv_ref):
    x = _combine_ln2(xprev_ref, eo0_ref, eo1_ref, s1p_ref, s2p_ref,
                     g2p_ref, b2p_ref)
    _core(x, pbin_ref[...], kw_ref, kb_ref, vw_ref, vb_ref, ow_ref, ob_ref,
          g1_ref, b1_ref, gw_ref, gb_ref,
          x1_ref, pos0_ref, pos1_ref, s1_ref, s2_ref, be_ref, nv_ref)


def _final_body(xprev_ref, eo0_ref, eo1_ref, s1p_ref, s2p_ref,
                g2p_ref, b2p_ref, out_ref):
    out_ref[...] = _combine_ln2(xprev_ref, eo0_ref, eo1_ref, s1p_ref, s2p_ref,
                                g2p_ref, b2p_ref)


def _expert_body(be_ref, nv_ref, xs_ref, ew1_ref, eb1_ref, ew2_ref, eb2_ref,
                 eo_ref):
    b = pl.program_id(0)

    @pl.when(nv_ref[b, 0] > 0)
    def _go():
        h = jnp.dot(xs_ref[...].astype(jnp.bfloat16),
                    ew1_ref[0].astype(jnp.bfloat16),
                    preferred_element_type=jnp.float32)
        h = jnp.maximum(h + eb1_ref[0], 0.0).astype(jnp.bfloat16)
        eo = jnp.dot(h, ew2_ref[0].astype(jnp.bfloat16),
                     preferred_element_type=jnp.float32) + eb2_ref[0]
        eo_ref[...] = eo


def _vm(shape=None):
    return pl.BlockSpec(memory_space=pltpu.VMEM)


def _a0_first_call(pbW1, pbb, pba, srcc, friT, frjT, cbfv, Wm, bm2,
                   kw, kb, vw, vb, ow, ob, g1, b1, gw, gb,
                   interpret):
    smem = pl.BlockSpec(memory_space=pltpu.SMEM)
    n_in = 19
    out_shape = [jax.ShapeDtypeStruct((_N, _D), jnp.float32),
                 jax.ShapeDtypeStruct((_N, 1), jnp.int32),
                 jax.ShapeDtypeStruct((_N, 1), jnp.int32),
                 jax.ShapeDtypeStruct((_N, 1), jnp.float32),
                 jax.ShapeDtypeStruct((_N, 1), jnp.float32),
                 jax.ShapeDtypeStruct((_NBLK, 1), jnp.int32),
                 jax.ShapeDtypeStruct((_NBLK, 1), jnp.int32),
                 jax.ShapeDtypeStruct((_N, _T), jnp.float32)]
    return pl.pallas_call(
        _a0_first_body,
        in_specs=[smem, smem, smem] + [_vm() for _ in range(n_in - 3)],
        out_specs=[_vm() for _ in out_shape],
        out_shape=out_shape,
        interpret=interpret,
    )(pbW1, pbb, pba, srcc, friT, frjT, cbfv, Wm, bm2,
      kw, kb, vw, vb, ow, ob, g1, b1, gw, gb)


def _a0_next_call(xprev, eo0, eo1, s1p, s2p, g2p, b2p, pbin,
                  kw, kb, vw, vb, ow, ob, g1, b1, gw, gb,
                  interpret):
    n_in = 18
    out_shape = [jax.ShapeDtypeStruct((_N, _D), jnp.float32),
                 jax.ShapeDtypeStruct((_N, 1), jnp.int32),
                 jax.ShapeDtypeStruct((_N, 1), jnp.int32),
                 jax.ShapeDtypeStruct((_N, 1), jnp.float32),
                 jax.ShapeDtypeStruct((_N, 1), jnp.float32),
                 jax.ShapeDtypeStruct((_NBLK, 1), jnp.int32),
                 jax.ShapeDtypeStruct((_NBLK, 1), jnp.int32)]
    return pl.pallas_call(
        _a0_next_body,
        in_specs=[_vm() for _ in range(n_in)],
        out_specs=[_vm() for _ in out_shape],
        out_shape=out_shape,
        interpret=interpret,
    )(xprev, eo0, eo1, s1p, s2p, g2p, b2p, pbin,
      kw, kb, vw, vb, ow, ob, g1, b1, gw, gb)


def _final_call(xprev, eo0, eo1, s1p, s2p, g2p, b2p, interpret):
    return pl.pallas_call(
        _final_body,
        in_specs=[_vm() for _ in range(7)],
        out_specs=_vm(),
        out_shape=jax.ShapeDtypeStruct((_N, _D), jnp.float32),
        interpret=interpret,
    )(xprev, eo0, eo1, s1p, s2p, g2p, b2p)


def _expert_call(be, nv, xs, ew1, eb1, ew2, eb2, interpret):
    grid_spec = pltpu.PrefetchScalarGridSpec(
        num_scalar_prefetch=2,
        grid=(_NBLK,),
        in_specs=[pl.BlockSpec((_G, _D), lambda b, be, nv: (b, 0)),
                  pl.BlockSpec((1, _D, _F), lambda b, be, nv: (be[b, 0], 0, 0)),
                  pl.BlockSpec((1, 1, _F), lambda b, be, nv: (be[b, 0], 0, 0)),
                  pl.BlockSpec((1, _F, _D), lambda b, be, nv: (be[b, 0], 0, 0)),
                  pl.BlockSpec((1, 1, _D), lambda b, be, nv: (be[b, 0], 0, 0))],
        out_specs=pl.BlockSpec((_G, _D), lambda b, be, nv: (b, 0)),
    )
    return pl.pallas_call(
        _expert_body,
        grid_spec=grid_spec,
        out_shape=jax.ShapeDtypeStruct((_NPAD, _D), jnp.float32),
        compiler_params=pltpu.CompilerParams(
            dimension_semantics=("arbitrary",)),
        interpret=interpret,
    )(be, nv, xs, ew1, eb1, ew2, eb2)


_TPW = _N // _NW
_RPW = 2 * _N // _NW
def _sc_scatter(x1b, pos0, pos1):
    mesh = plsc.VectorSubcoreMesh(core_axis_name="c", subcore_axis_name="s")

    @functools.partial(
        pl.kernel, mesh=mesh,
        out_type=jax.ShapeDtypeStruct((_NPAD, _D), jnp.float32),
        scratch_types=[pltpu.VMEM((_TPW,), jnp.int32),
                       pltpu.VMEM((_TPW,), jnp.int32),
                       pltpu.VMEM((_TPW, _D), jnp.float32),
                       pltpu.SemaphoreType.DMA,
                       pltpu.SemaphoreType.DMA])
    def k(x_hbm, p0_hbm, p1_hbm, xs_hbm, i0_v, i1_v, rows_v, sem0, sem1):
        wid = lax.axis_index("s") * 2 + lax.axis_index("c")
        base = wid * _TPW
        pltpu.sync_copy(p0_hbm.at[pl.ds(base, _TPW)], i0_v)
        pltpu.sync_copy(p1_hbm.at[pl.ds(base, _TPW)], i1_v)
        pltpu.sync_copy(x_hbm.at[pl.ds(base, _TPW)], rows_v)
        c0 = pltpu.async_copy(rows_v, xs_hbm.at[i0_v], sem0)
        c1 = pltpu.async_copy(rows_v, xs_hbm.at[i1_v], sem1)
        c0.wait()
        c1.wait()

    return k(x1b, pos0, pos1)


def _sc_gather(eo, pos0, pos1):
    mesh = plsc.VectorSubcoreMesh(core_axis_name="c", subcore_axis_name="s")

    @functools.partial(
        pl.kernel, mesh=mesh,
        out_type=[jax.ShapeDtypeStruct((_N, _D), jnp.float32),
                  jax.ShapeDtypeStruct((_N, _D), jnp.float32)],
        scratch_types=[pltpu.VMEM((_TPW,), jnp.int32),
                       pltpu.VMEM((_TPW,), jnp.int32),
                       pltpu.VMEM((_TPW, _D), jnp.float32),
                       pltpu.VMEM((_TPW, _D), jnp.float32),
                       pltpu.SemaphoreType.DMA,
                       pltpu.SemaphoreType.DMA])
    def k(eo_hbm, p0_hbm, p1_hbm, o0_hbm, o1_hbm,
          i0_v, i1_v, r0_v, r1_v, sem0, sem1):
        wid = lax.axis_index("s") * 2 + lax.axis_index("c")
        base = wid * _TPW
        pltpu.sync_copy(p0_hbm.at[pl.ds(base, _TPW)], i0_v)
        pltpu.sync_copy(p1_hbm.at[pl.ds(base, _TPW)], i1_v)
        c0 = pltpu.async_copy(eo_hbm.at[i0_v], r0_v, sem0)
        c1 = pltpu.async_copy(eo_hbm.at[i1_v], r1_v, sem1)
        c0.wait()
        c1.wait()
        pltpu.sync_copy(r0_v, o0_hbm.at[pl.ds(base, _TPW)])
        pltpu.sync_copy(r1_v, o1_hbm.at[pl.ds(base, _TPW)])

    return k(eo, pos0, pos1)


def kernel(src, frac, cbfv, Wm, bm, pbW, pbb, pba, key_w, key_b, val_w, val_b,
           out_w, out_b, ln1_g, ln1_b, gate_w, gate_b, e_w1, e_b1, e_w2, e_b2,
           ln2_g, ln2_b, *, interpret=False):
    srcc = src.reshape(_N, 1).astype(jnp.int32)
    friT = frac.reshape(1, _N)
    frjT = jnp.repeat(frac.T, _T, axis=1)
    kb = key_b.reshape(_NL, 1, _D); vb = val_b.reshape(_NL, 1, _D)
    ob = out_b.reshape(_NL, 1, _D)
    g1 = ln1_g.reshape(_NL, 1, _D); b1 = ln1_b.reshape(_NL, 1, _D)
    g2 = ln2_g.reshape(_NL, 1, _D); b2 = ln2_b.reshape(_NL, 1, _D)
    gb = gate_b.reshape(_NL, 1, _E)
    eb1 = e_b1.reshape(_NL, _E, 1, _F); eb2 = e_b2.reshape(_NL, _E, 1, _D)

    x1, pos0, pos1, s1, s2, be, nv, pb = _a0_first_call(
        pbW.reshape(_PBF), pbb, pba, srcc, friT, frjT, cbfv, Wm,
        bm.reshape(1, _D), key_w[0], kb[0], val_w[0], vb[0], out_w[0], ob[0],
        g1[0], b1[0], gate_w[0], gb[0], interpret)
    xs = _sc_scatter(x1, pos0.reshape(_N), pos1.reshape(_N))
    eo = _expert_call(be, nv, xs, e_w1[0], eb1[0], e_w2[0], eb2[0], interpret)
    eo0, eo1 = _sc_gather(eo, pos0.reshape(_N), pos1.reshape(_N))

    x1n, pos0n, pos1n, s1n, s2n, ben, nvn = _a0_next_call(
        x1, eo0, eo1, s1, s2, g2[0], b2[0], pb,
        key_w[1], kb[1], val_w[1], vb[1], out_w[1], ob[1],
        g1[1], b1[1], gate_w[1], gb[1], interpret)
    xsn = _sc_scatter(x1n, pos0n.reshape(_N), pos1n.reshape(_N))
    eon = _expert_call(ben, nvn, xsn, e_w1[1], eb1[1], e_w2[1], eb2[1],
                       interpret)
    eo0n, eo1n = _sc_gather(eon, pos0n.reshape(_N), pos1n.reshape(_N))

    out = _final_call(x1n, eo0n, eo1n, s1n, s2n, g2[1], b2[1], interpret)
    return out.reshape(_B, _T, _D)


# speedup vs baseline: 1.1207x; 1.1207x over previous
import functools

import jax
import jax.numpy as jnp
from jax import lax
from jax.experimental import pallas as pl
from jax.experimental.pallas import tpu as pltpu
from jax.experimental.pallas import tpu_sc as plsc

_B = 256; _T = 8; _D = 512; _NL = 2; _E = 8; _F = 1024
_VOCAB = 119; _FEAT = 200; _PBF = 16; _PB_SCALE = 10.0
_N = _B * _T
_G = 256
_NBLK = 24
_NPAD = _NBLK * _G
_NW = 32


def _ln_rows(x, g, b):
    mu = jnp.mean(x, axis=-1, keepdims=True)
    xc = x - mu
    var = jnp.mean(xc * xc, axis=-1, keepdims=True)
    return xc * jax.lax.rsqrt(var + 1e-5) * g + b


def _core(x, pbv, kw_ref, kb_ref, vw_ref, vb_ref, ow_ref, ob_ref,
          g1_ref, b1_ref, gw_ref, gb_ref,
          x1_ref, pos0_ref, pos1_ref, s1_ref, s2_ref, be_ref, nv_ref):
    xb = x.astype(jnp.bfloat16)
    K = jnp.dot(xb, kw_ref[...], preferred_element_type=jnp.float32) + kb_ref[...]
    V = jnp.dot(xb, vw_ref[...], preferred_element_type=jnp.float32) + vb_ref[...]
    K3 = K.reshape(_B, _T, _D)
    V3 = V.reshape(_B, _T, _D)
    pb3 = pbv.reshape(_B, _T, _T)
    wv3 = jnp.broadcast_to(jnp.sum(K3 * V3, axis=1, keepdims=True), (_B, _T, _D))
    for jj in range(_T):
        wv3 = wv3 + pb3[:, :, jj:jj + 1] * V3[:, jj:jj + 1, :]
    wv = wv3.reshape(_N, _D).astype(jnp.bfloat16)
    attn = jnp.dot(wv, ow_ref[...], preferred_element_type=jnp.float32) + ob_ref[...]
    xn = _ln_rows(x + attn, g1_ref[...], b1_ref[...])
    x1_ref[...] = xn

    logits = jnp.dot(xn, gw_ref[...], preferred_element_type=jnp.float32) + gb_ref[...]
    m = jnp.max(logits, axis=-1, keepdims=True)
    ex = jnp.exp(logits - m)
    p = ex / jnp.sum(ex, axis=-1, keepdims=True)
    lane = jax.lax.broadcasted_iota(jnp.int32, (_N, _E), 1)
    m1 = jnp.max(p, axis=-1, keepdims=True)
    i1 = jnp.min(jnp.where(p == m1, lane, _E), axis=-1, keepdims=True)
    oh1 = lane == i1
    pm = jnp.where(oh1, -1.0, p)
    m2 = jnp.max(pm, axis=-1, keepdims=True)
    i2 = jnp.min(jnp.where(pm == m2, lane, _E), axis=-1, keepdims=True)
    oh2 = lane == i2
    s1_ref[...] = m1
    s2_ref[...] = m2

    m0f = jnp.where(oh1, 1.0, 0.0)
    m1f = jnp.where(oh2, 1.0, 0.0)
    mm = jnp.concatenate([m0f, m1f], axis=1).astype(jnp.bfloat16)
    tri = (jax.lax.broadcasted_iota(jnp.int32, (_N, _N), 0)
           >= jax.lax.broadcasted_iota(jnp.int32, (_N, _N), 1)).astype(jnp.bfloat16)
    cum = jnp.dot(tri, mm, preferred_element_type=jnp.float32)
    cum0 = cum[:, :_E]
    cum1 = cum[:, _E:]
    c0 = cum0[_N - 1:_N, :]
    ct = c0 + cum1[_N - 1:_N, :]
    cti = ct.astype(jnp.int32)
    cpad = (jax.lax.shift_right_logical(cti + (_G - 1), 8) * _G).astype(jnp.float32)
    e0 = jax.lax.broadcasted_iota(jnp.int32, (_E, _E), 0)
    e1 = jax.lax.broadcasted_iota(jnp.int32, (_E, _E), 1)
    tri8x = (e0 < e1).astype(jnp.float32)
    tri8i = (e0 <= e1).astype(jnp.float32)
    bs = jnp.dot(cpad, tri8x, preferred_element_type=jnp.float32)
    pos0 = jnp.sum(m0f * (cum0 - 1.0 + bs), axis=-1, keepdims=True)
    pos1 = jnp.sum(m1f * (cum1 - 1.0 + bs + c0), axis=-1, keepdims=True)
    pos0_ref[...] = pos0.astype(jnp.int32)
    pos1_ref[...] = pos1.astype(jnp.int32)

    bcnt = cpad * (1.0 / _G)
    ib = jnp.dot(bcnt, tri8i, preferred_element_type=jnp.float32)
    ba = jax.lax.broadcasted_iota(jnp.int32, (_NBLK, 1), 0).astype(jnp.float32)
    ge = (ba >= jnp.broadcast_to(ib, (_NBLK, _E))).astype(jnp.float32)
    be = jnp.sum(ge, axis=-1, keepdims=True)
    be_i = jnp.minimum(be, float(_E - 1)).astype(jnp.int32)
    lane8 = jax.lax.broadcasted_iota(jnp.int32, (_NBLK, _E), 1)
    ohbe = lane8 == be_i
    startb = ib - bcnt
    sbe = jnp.sum(jnp.where(ohbe, jnp.broadcast_to(startb, (_NBLK, _E)), 0.0),
                  axis=-1, keepdims=True)
    cbe = jnp.sum(jnp.where(ohbe, jnp.broadcast_to(ct, (_NBLK, _E)), 0.0),
                  axis=-1, keepdims=True)
    nv = jnp.clip(cbe - (ba - sbe) * _G, 0.0, float(_G))
    nv = jnp.where(be >= float(_E), 0.0, nv)
    be_ref[...] = be_i
    nv_ref[...] = nv.astype(jnp.int32)


def _combine_ln2(xprev_ref, eo0_ref, eo1_ref, s1p_ref, s2p_ref, g2_ref, b2_ref):
    moe = s1p_ref[...] * eo0_ref[...] + s2p_ref[...] * eo1_ref[...]
    return _ln_rows(xprev_ref[...] + moe, g2_ref[...], b2_ref[...])


def _a0_first_body(pbW_s, pbb_s, pba_s,
                   src_ref, friT_ref, frjT_ref, cbfv_ref, Wm_ref, bm_ref,
                   kw_ref, kb_ref, vw_ref, vb_ref, ow_ref, ob_ref,
                   g1_ref, b1_ref, gw_ref, gb_ref,
                   x1_ref, pos0_ref, pos1_ref, s1_ref, s2_ref,
                   be_ref, nv_ref, pb_ref):
    cw = jnp.dot(cbfv_ref[...], Wm_ref[...], preferred_element_type=jnp.float32)
    oh = (src_ref[...] == jax.lax.broadcasted_iota(jnp.int32, (_N, _VOCAB), 1)
          ).astype(jnp.float32)
    x = jnp.dot(oh, cw, preferred_element_type=jnp.float32) + bm_ref[...]
    dT = (frjT_ref[...] - friT_ref[...]) * _PB_SCALE
    acc = jnp.zeros((_T, _N), jnp.float32)
    for f in range(_PBF):
        acc = acc + jnp.cos(dT * pbW_s[f] + pbb_s[f]) * pba_s[f]
    pbv = acc.T
    pb_ref[...] = pbv
    _core(x, pbv, kw_ref, kb_ref, vw_ref, vb_ref, ow_ref, ob_ref,
          g1_ref, b1_ref, gw_ref, gb_ref,
          x1_ref, pos0_ref, pos1_ref, s1_ref, s2_ref, be_ref, nv_ref)


def _a0_next_body(xprev_ref, eo0_ref, eo1_ref, s1p_ref, s2p_ref,
                  g2p_ref, b2p_ref, pbin_ref,
                  kw_ref, kb_ref, vw_ref, vb_ref, ow_ref, ob_ref,
                  g1_ref, b1_ref, gw_ref, gb_ref,
                  x1_ref, pos0_ref, pos1_ref, s1_ref, s2_ref,
                  be_ref, nv_ref):
    x = _combine_ln2(xprev_ref, eo0_ref, eo1_ref, s1p_ref, s2p_ref,
                     g2p_ref, b2p_ref)
    _core(x, pbin_ref[...], kw_ref, kb_ref, vw_ref, vb_ref, ow_ref, ob_ref,
          g1_ref, b1_ref, gw_ref, gb_ref,
          x1_ref, pos0_ref, pos1_ref, s1_ref, s2_ref, be_ref, nv_ref)


def _final_body(xprev_ref, eo0_ref, eo1_ref, s1p_ref, s2p_ref,
                g2p_ref, b2p_ref, out_ref):
    out_ref[...] = _combine_ln2(xprev_ref, eo0_ref, eo1_ref, s1p_ref, s2p_ref,
                                g2p_ref, b2p_ref)


def _expert_body(be_ref, nv_ref, xs_ref, ew1_ref, eb1_ref, ew2_ref, eb2_ref,
                 eo_ref):
    b = pl.program_id(0)

    @pl.when(nv_ref[b, 0] > 0)
    def _go():
        h = jnp.dot(xs_ref[...].astype(jnp.bfloat16), ew1_ref[0],
                    preferred_element_type=jnp.float32)
        h = jnp.maximum(h + eb1_ref[0], 0.0).astype(jnp.bfloat16)
        eo = jnp.dot(h, ew2_ref[0], preferred_element_type=jnp.float32) + eb2_ref[0]
        eo_ref[...] = eo


def _vm(shape=None):
    return pl.BlockSpec(memory_space=pltpu.VMEM)


def _a0_first_call(pbW1, pbb, pba, srcc, friT, frjT, cbfv, Wm, bm2,
                   kw, kb, vw, vb, ow, ob, g1, b1, gw, gb,
                   interpret):
    smem = pl.BlockSpec(memory_space=pltpu.SMEM)
    n_in = 19
    out_shape = [jax.ShapeDtypeStruct((_N, _D), jnp.float32),
                 jax.ShapeDtypeStruct((_N, 1), jnp.int32),
                 jax.ShapeDtypeStruct((_N, 1), jnp.int32),
                 jax.ShapeDtypeStruct((_N, 1), jnp.float32),
                 jax.ShapeDtypeStruct((_N, 1), jnp.float32),
                 jax.ShapeDtypeStruct((_NBLK, 1), jnp.int32),
                 jax.ShapeDtypeStruct((_NBLK, 1), jnp.int32),
                 jax.ShapeDtypeStruct((_N, _T), jnp.float32)]
    return pl.pallas_call(
        _a0_first_body,
        in_specs=[smem, smem, smem] + [_vm() for _ in range(n_in - 3)],
        out_specs=[_vm() for _ in out_shape],
        out_shape=out_shape,
        interpret=interpret,
    )(pbW1, pbb, pba, srcc, friT, frjT, cbfv, Wm, bm2,
      kw, kb, vw, vb, ow, ob, g1, b1, gw, gb)


def _a0_next_call(xprev, eo0, eo1, s1p, s2p, g2p, b2p, pbin,
                  kw, kb, vw, vb, ow, ob, g1, b1, gw, gb,
                  interpret):
    n_in = 18
    out_shape = [jax.ShapeDtypeStruct((_N, _D), jnp.float32),
                 jax.ShapeDtypeStruct((_N, 1), jnp.int32),
                 jax.ShapeDtypeStruct((_N, 1), jnp.int32),
                 jax.ShapeDtypeStruct((_N, 1), jnp.float32),
                 jax.ShapeDtypeStruct((_N, 1), jnp.float32),
                 jax.ShapeDtypeStruct((_NBLK, 1), jnp.int32),
                 jax.ShapeDtypeStruct((_NBLK, 1), jnp.int32)]
    return pl.pallas_call(
        _a0_next_body,
        in_specs=[_vm() for _ in range(n_in)],
        out_specs=[_vm() for _ in out_shape],
        out_shape=out_shape,
        interpret=interpret,
    )(xprev, eo0, eo1, s1p, s2p, g2p, b2p, pbin,
      kw, kb, vw, vb, ow, ob, g1, b1, gw, gb)


def _final_call(xprev, eo0, eo1, s1p, s2p, g2p, b2p, interpret):
    return pl.pallas_call(
        _final_body,
        in_specs=[_vm() for _ in range(7)],
        out_specs=_vm(),
        out_shape=jax.ShapeDtypeStruct((_N, _D), jnp.float32),
        interpret=interpret,
    )(xprev, eo0, eo1, s1p, s2p, g2p, b2p)


def _expert_call(be, nv, xs, ew1, eb1, ew2, eb2, interpret):
    grid_spec = pltpu.PrefetchScalarGridSpec(
        num_scalar_prefetch=2,
        grid=(_NBLK,),
        in_specs=[pl.BlockSpec((_G, _D), lambda b, be, nv: (b, 0)),
                  pl.BlockSpec((1, _D, _F), lambda b, be, nv: (be[b, 0], 0, 0)),
                  pl.BlockSpec((1, 1, _F), lambda b, be, nv: (be[b, 0], 0, 0)),
                  pl.BlockSpec((1, _F, _D), lambda b, be, nv: (be[b, 0], 0, 0)),
                  pl.BlockSpec((1, 1, _D), lambda b, be, nv: (be[b, 0], 0, 0))],
        out_specs=pl.BlockSpec((_G, _D), lambda b, be, nv: (b, 0)),
    )
    return pl.pallas_call(
        _expert_body,
        grid_spec=grid_spec,
        out_shape=jax.ShapeDtypeStruct((_NPAD, _D), jnp.float32),
        compiler_params=pltpu.CompilerParams(
            dimension_semantics=("arbitrary",)),
        interpret=interpret,
    )(be, nv, xs, ew1, eb1, ew2, eb2)


_TPW = _N // _NW
_RPW = 2 * _N // _NW
def _sc_scatter(x1b, pos0, pos1):
    mesh = plsc.VectorSubcoreMesh(core_axis_name="c", subcore_axis_name="s")

    @functools.partial(
        pl.kernel, mesh=mesh,
        out_type=jax.ShapeDtypeStruct((_NPAD, _D), jnp.float32),
        scratch_types=[pltpu.VMEM((_TPW,), jnp.int32),
                       pltpu.VMEM((_TPW,), jnp.int32),
                       pltpu.VMEM((_TPW, _D), jnp.float32),
                       pltpu.SemaphoreType.DMA,
                       pltpu.SemaphoreType.DMA])
    def k(x_hbm, p0_hbm, p1_hbm, xs_hbm, i0_v, i1_v, rows_v, sem0, sem1):
        wid = lax.axis_index("s") * 2 + lax.axis_index("c")
        base = wid * _TPW
        pltpu.sync_copy(p0_hbm.at[pl.ds(base, _TPW)], i0_v)
        pltpu.sync_copy(p1_hbm.at[pl.ds(base, _TPW)], i1_v)
        pltpu.sync_copy(x_hbm.at[pl.ds(base, _TPW)], rows_v)
        c0 = pltpu.async_copy(rows_v, xs_hbm.at[i0_v], sem0)
        c1 = pltpu.async_copy(rows_v, xs_hbm.at[i1_v], sem1)
        c0.wait()
        c1.wait()

    return k(x1b, pos0, pos1)


def _sc_gather(eo, pos0, pos1):
    mesh = plsc.VectorSubcoreMesh(core_axis_name="c", subcore_axis_name="s")

    @functools.partial(
        pl.kernel, mesh=mesh,
        out_type=[jax.ShapeDtypeStruct((_N, _D), jnp.float32),
                  jax.ShapeDtypeStruct((_N, _D), jnp.float32)],
        scratch_types=[pltpu.VMEM((_TPW,), jnp.int32),
                       pltpu.VMEM((_TPW,), jnp.int32),
                       pltpu.VMEM((_TPW, _D), jnp.float32),
                       pltpu.VMEM((_TPW, _D), jnp.float32),
                       pltpu.SemaphoreType.DMA,
                       pltpu.SemaphoreType.DMA])
    def k(eo_hbm, p0_hbm, p1_hbm, o0_hbm, o1_hbm,
          i0_v, i1_v, r0_v, r1_v, sem0, sem1):
        wid = lax.axis_index("s") * 2 + lax.axis_index("c")
        base = wid * _TPW
        pltpu.sync_copy(p0_hbm.at[pl.ds(base, _TPW)], i0_v)
        pltpu.sync_copy(p1_hbm.at[pl.ds(base, _TPW)], i1_v)
        c0 = pltpu.async_copy(eo_hbm.at[i0_v], r0_v, sem0)
        c1 = pltpu.async_copy(eo_hbm.at[i1_v], r1_v, sem1)
        c0.wait()
        c1.wait()
        pltpu.sync_copy(r0_v, o0_hbm.at[pl.ds(base, _TPW)])
        pltpu.sync_copy(r1_v, o1_hbm.at[pl.ds(base, _TPW)])

    return k(eo, pos0, pos1)


def kernel(src, frac, cbfv, Wm, bm, pbW, pbb, pba, key_w, key_b, val_w, val_b,
           out_w, out_b, ln1_g, ln1_b, gate_w, gate_b, e_w1, e_b1, e_w2, e_b2,
           ln2_g, ln2_b, *, interpret=False):
    bf = jnp.bfloat16
    srcc = src.reshape(_N, 1).astype(jnp.int32)
    friT = frac.reshape(1, _N)
    frjT = jnp.repeat(frac.T, _T, axis=1)
    key_w = key_w.astype(bf); val_w = val_w.astype(bf); out_w = out_w.astype(bf)
    e_w1 = e_w1.astype(bf); e_w2 = e_w2.astype(bf)
    kb = key_b.reshape(_NL, 1, _D); vb = val_b.reshape(_NL, 1, _D)
    ob = out_b.reshape(_NL, 1, _D)
    g1 = ln1_g.reshape(_NL, 1, _D); b1 = ln1_b.reshape(_NL, 1, _D)
    g2 = ln2_g.reshape(_NL, 1, _D); b2 = ln2_b.reshape(_NL, 1, _D)
    gb = gate_b.reshape(_NL, 1, _E)
    eb1 = e_b1.reshape(_NL, _E, 1, _F); eb2 = e_b2.reshape(_NL, _E, 1, _D)

    x1, pos0, pos1, s1, s2, be, nv, pb = _a0_first_call(
        pbW.reshape(_PBF), pbb, pba, srcc, friT, frjT, cbfv, Wm,
        bm.reshape(1, _D), key_w[0], kb[0], val_w[0], vb[0], out_w[0], ob[0],
        g1[0], b1[0], gate_w[0], gb[0], interpret)
    xs = _sc_scatter(x1, pos0.reshape(_N), pos1.reshape(_N))
    eo = _expert_call(be, nv, xs, e_w1[0], eb1[0], e_w2[0], eb2[0], interpret)
    eo0, eo1 = _sc_gather(eo, pos0.reshape(_N), pos1.reshape(_N))

    x1n, pos0n, pos1n, s1n, s2n, ben, nvn = _a0_next_call(
        x1, eo0, eo1, s1, s2, g2[0], b2[0], pb,
        key_w[1], kb[1], val_w[1], vb[1], out_w[1], ob[1],
        g1[1], b1[1], gate_w[1], gb[1], interpret)
    xsn = _sc_scatter(x1n, pos0n.reshape(_N), pos1n.reshape(_N))
    eon = _expert_call(ben, nvn, xsn, e_w1[1], eb1[1], e_w2[1], eb2[1],
                       interpret)
    eo0n, eo1n = _sc_gather(eon, pos0n.reshape(_N), pos1n.reshape(_N))

    out = _final_call(x1n, eo0n, eo1n, s1n, s2n, g2[1], b2[1], interpret)
    return out.reshape(_B, _T, _D)
